# baseline (speedup 1.0000x reference)
import functools

import jax
import jax.numpy as jnp
from jax import lax
from jax.experimental import pallas as pl
from jax.experimental.pallas import tpu as pltpu
from jax.experimental.pallas import tpu_sc as plsc

H = 768
F = 3072
E = 8
T = 2048
TB = 512
BT = 256
NB = 16
NSLOT = NB * BT
PW = 128
NC, NS = 2, 16
NW = NC * NS
TPW = T // NW


_NSTEP = T // TB + 1


def _router_body(x_ref, wr_ref, logits_ref, pmax_ref, counts_ref, psum_ref,
                 pos_ref, bexp_ref, bval_ref, carry_ref, routes_s, rank_s):
    i = pl.program_id(0)

    @pl.when(i == 0)
    def _():
        counts_ref[...] = jnp.zeros_like(counts_ref)
        psum_ref[...] = jnp.zeros_like(psum_ref)
        carry_ref[...] = jnp.zeros_like(carry_ref)

    @pl.when(i < _NSTEP - 1)
    def _():
        xb = x_ref[...]
        logits = jnp.dot(xb, wr_ref[...], preferred_element_type=jnp.float32)
        logits_ref[...] = logits
        m = jnp.max(logits, axis=1, keepdims=True)
        ex = jnp.exp(logits - m)
        p = ex / jnp.sum(ex, axis=1, keepdims=True)
        pmax = jnp.max(p, axis=1, keepdims=True)
        iota_e = lax.broadcasted_iota(jnp.int32, (TB, E), 1)
        routes = jnp.min(jnp.where(p == pmax, iota_e, E), axis=1)
        mask = (iota_e == routes[:, None]).astype(jnp.float32)

        ltri = (lax.broadcasted_iota(jnp.int32, (TB, TB), 0)
                > lax.broadcasted_iota(jnp.int32, (TB, TB), 1)
                ).astype(jnp.float32)
        exc = jnp.dot(ltri, mask, preferred_element_type=jnp.float32)
        rank = jnp.sum(mask * (carry_ref[...] + exc), axis=1)
        routes_s[pl.ds(i * TB, TB)] = routes
        rank_s[pl.ds(i * TB, TB)] = rank.astype(jnp.int32)

        counts_ref[...] += jnp.sum(mask, axis=0, keepdims=True)
        psum_ref[...] += jnp.sum(p, axis=0, keepdims=True)
        carry_ref[...] += jnp.sum(mask, axis=0, keepdims=True)

        pmax_ref[...] = jnp.broadcast_to(pmax, (TB, PW))

    @pl.when(i == _NSTEP - 1)
    def _():
        counts = counts_ref[...]
        nblk = jnp.floor((counts + (BT - 1)) * (1.0 / BT))
        ilt = (lax.broadcasted_iota(jnp.int32, (E, E), 0)
               < lax.broadcasted_iota(jnp.int32, (E, E), 1)).astype(jnp.float32)
        ile = (lax.broadcasted_iota(jnp.int32, (E, E), 0)
               <= lax.broadcasted_iota(jnp.int32, (E, E), 1)).astype(jnp.float32)
        off = jnp.dot(nblk * BT, ilt, preferred_element_type=jnp.float32)
        bend = jnp.dot(nblk, ile, preferred_element_type=jnp.float32)
        total = bend[0, E - 1]

        routes = routes_s[...]
        maskf = (lax.broadcasted_iota(jnp.int32, (T, E), 1)
                 == routes[:, None]).astype(jnp.float32)
        base = jnp.sum(maskf * off, axis=1)
        pos_ref[...] = rank_s[...] + base.astype(jnp.int32)

        biota = lax.broadcasted_iota(jnp.int32, (NB, E), 0).astype(jnp.float32)
        bexp = jnp.sum((biota >= jnp.broadcast_to(bend, (NB, E))
                        ).astype(jnp.float32), axis=1)
        bexp_ref[...] = jnp.minimum(bexp, E - 1).astype(jnp.int32)
        bval_ref[...] = (biota[:, 0] < total).astype(jnp.int32)


def _run_router(xf, W_r):
    nc = T // TB - 1
    return pl.pallas_call(
        _router_body,
        grid=(_NSTEP,),
        in_specs=[
            pl.BlockSpec((TB, H), lambda i: (jnp.minimum(i, nc), 0)),
            pl.BlockSpec((H, E), lambda i: (0, 0)),
        ],
        out_specs=[
            pl.BlockSpec((TB, E), lambda i: (jnp.minimum(i, nc), 0)),
            pl.BlockSpec((TB, PW), lambda i: (jnp.minimum(i, nc), 0)),
            pl.BlockSpec((1, E), lambda i: (0, 0)),
            pl.BlockSpec((1, E), lambda i: (0, 0)),
            pl.BlockSpec((T,), lambda i: (0,)),
            pl.BlockSpec((NB,), lambda i: (0,)),
            pl.BlockSpec((NB,), lambda i: (0,)),
        ],
        out_shape=[
            jax.ShapeDtypeStruct((T, E), jnp.float32),
            jax.ShapeDtypeStruct((T, PW), jnp.float32),
            jax.ShapeDtypeStruct((1, E), jnp.float32),
            jax.ShapeDtypeStruct((1, E), jnp.float32),
            jax.ShapeDtypeStruct((T,), jnp.int32),
            jax.ShapeDtypeStruct((NB,), jnp.int32),
            jax.ShapeDtypeStruct((NB,), jnp.int32),
        ],
        scratch_shapes=[
            pltpu.VMEM((1, E), jnp.float32),
            pltpu.VMEM((T,), jnp.int32),
            pltpu.VMEM((T,), jnp.int32),
        ],
    )(xf, W_r)


@functools.cache
def _sc_dispatch():
    mesh = plsc.VectorSubcoreMesh(
        core_axis_name="c", subcore_axis_name="s",
        num_cores=NC, num_subcores=NS)

    @functools.partial(
        pl.kernel,
        mesh=mesh,
        out_type=[
            jax.ShapeDtypeStruct((NSLOT, H), jnp.float32),
            jax.ShapeDtypeStruct((NSLOT, PW), jnp.float32),
        ],
        scratch_types=[
            pltpu.VMEM((TPW,), jnp.int32),
            pltpu.VMEM((TPW, H), jnp.float32),
            pltpu.VMEM((TPW, PW), jnp.float32),
            pltpu.SemaphoreType.DMA,
            pltpu.SemaphoreType.DMA,
        ],
    )
    def dispatch(x_hbm, pmax_hbm, pos_hbm, xs_hbm, ps_hbm, idx_v, rows_v,
                 pv, sem, sem2):
        wid = lax.axis_index("s") * NC + lax.axis_index("c")
        base = wid * TPW
        pltpu.sync_copy(pos_hbm.at[pl.ds(base, TPW)], idx_v)
        pltpu.sync_copy(x_hbm.at[pl.ds(base, TPW)], rows_v)
        pltpu.sync_copy(pmax_hbm.at[pl.ds(base, TPW)], pv)
        row_copy = pltpu.async_copy(rows_v, xs_hbm.at[idx_v], sem)
        p_copy = pltpu.async_copy(pv, ps_hbm.at[idx_v], sem2)
        row_copy.wait()
        p_copy.wait()

    return dispatch


def _mlp_body(bexp_ref, bval_ref, xs_ref, ps_ref, w1_ref, b1_ref, w2_ref,
              b2_ref, out_ref):
    b = pl.program_id(0)

    @pl.when(bval_ref[b] == 1)
    def _():
        e = bexp_ref[b]
        xb = xs_ref[...].astype(jnp.bfloat16)
        pmax = ps_ref[:, :1]
        h = jnp.dot(xb, w1_ref[0], preferred_element_type=jnp.float32)
        h = jax.nn.gelu(h + b1_ref[pl.ds(e, 1), :])
        o = jnp.dot(h.astype(jnp.bfloat16), w2_ref[0],
                    preferred_element_type=jnp.float32)
        out_ref[...] = (o + b2_ref[pl.ds(e, 1), :]) * pmax


def _run_mlp(bexp, bval, xs, ps, W1, b1, W2, b2):
    grid_spec = pltpu.PrefetchScalarGridSpec(
        num_scalar_prefetch=2,
        grid=(NB,),
        in_specs=[
            pl.BlockSpec((BT, H), lambda b, be, bv: (b, 0)),
            pl.BlockSpec((BT, PW), lambda b, be, bv: (b, 0)),
            pl.BlockSpec((1, H, F), lambda b, be, bv: (be[b], 0, 0)),
            pl.BlockSpec((E, F), lambda b, be, bv: (0, 0)),
            pl.BlockSpec((1, F, H), lambda b, be, bv: (be[b], 0, 0)),
            pl.BlockSpec((E, H), lambda b, be, bv: (0, 0)),
        ],
        out_specs=pl.BlockSpec((BT, H), lambda b, be, bv: (b, 0)),
    )
    return pl.pallas_call(
        _mlp_body,
        grid_spec=grid_spec,
        out_shape=jax.ShapeDtypeStruct((NSLOT, H), jnp.float32),
        compiler_params=pltpu.CompilerParams(
            vmem_limit_bytes=120 * 1024 * 1024),
    )(bexp, bval, xs, ps, W1, b1, W2, b2)


@functools.cache
def _sc_combine():
    mesh = plsc.VectorSubcoreMesh(
        core_axis_name="c", subcore_axis_name="s",
        num_cores=NC, num_subcores=NS)

    @functools.partial(
        pl.kernel,
        mesh=mesh,
        out_type=jax.ShapeDtypeStruct((T, H), jnp.float32),
        scratch_types=[
            pltpu.VMEM((TPW,), jnp.int32),
            pltpu.VMEM((TPW, H), jnp.float32),
            pltpu.SemaphoreType.DMA,
        ],
    )
    def combine(ys_hbm, pos_hbm, y_hbm, idx_v, rows_v, sem):
        wid = lax.axis_index("s") * NC + lax.axis_index("c")
        base = wid * TPW
        pltpu.sync_copy(pos_hbm.at[pl.ds(base, TPW)], idx_v)
        pltpu.async_copy(ys_hbm.at[idx_v], rows_v, sem).wait()
        pltpu.sync_copy(rows_v, y_hbm.at[pl.ds(base, TPW)])

    return combine


@jax.jit
def kernel(x, W_r, W1, b1, W2, b2):
    original_shape = x.shape
    xf = x.reshape(T, H)
    logits, pmax16, counts, psum, pos, bexp, bval = _run_router(xf, W_r)
    xs, ps = _sc_dispatch()(xf, pmax16, pos)
    ys = _run_mlp(bexp, bval, xs, ps,
                  W1.astype(jnp.bfloat16), b1,
                  W2.astype(jnp.bfloat16), b2)
    y = _sc_combine()(ys, pos)
    return (y.reshape(original_shape), counts[0], psum[0],
            logits.reshape(original_shape[:-1] + (E,)))


# speedup vs baseline: 1.2736x; 1.2736x over previous
import functools

import jax
import jax.numpy as jnp
from jax import lax
from jax.experimental import pallas as pl
from jax.experimental.pallas import tpu as pltpu
from jax.experimental.pallas import tpu_sc as plsc

H = 768
F = 3072
E = 8
T = 2048
TB = 512
BT = 128
NB = 24
NSLOT = NB * BT
PW = 128
NC, NS = 2, 16
NW = NC * NS
TPW = T // NW


_NSTEP = T // TB + 1


def _router_body(x_ref, wr_ref, logits_ref, pmax_ref, counts_ref, psum_ref,
                 pos_ref, bexp_ref, bval_ref, carry_ref, routes_s, rank_s):
    i = pl.program_id(0)

    @pl.when(i == 0)
    def _():
        counts_ref[...] = jnp.zeros_like(counts_ref)
        psum_ref[...] = jnp.zeros_like(psum_ref)
        carry_ref[...] = jnp.zeros_like(carry_ref)

    @pl.when(i < _NSTEP - 1)
    def _():
        xb = x_ref[...]
        logits = jnp.dot(xb, wr_ref[...], preferred_element_type=jnp.float32)
        logits_ref[...] = logits
        m = jnp.max(logits, axis=1, keepdims=True)
        ex = jnp.exp(logits - m)
        p = ex / jnp.sum(ex, axis=1, keepdims=True)
        pmax = jnp.max(p, axis=1, keepdims=True)
        iota_e = lax.broadcasted_iota(jnp.int32, (TB, E), 1)
        routes = jnp.min(jnp.where(p == pmax, iota_e, E), axis=1)
        mask = (iota_e == routes[:, None]).astype(jnp.float32)

        ltri = (lax.broadcasted_iota(jnp.int32, (TB, TB), 0)
                > lax.broadcasted_iota(jnp.int32, (TB, TB), 1)
                ).astype(jnp.float32)
        exc = jnp.dot(ltri, mask, preferred_element_type=jnp.float32)
        rank = jnp.sum(mask * (carry_ref[...] + exc), axis=1)
        routes_s[pl.ds(i * TB, TB)] = routes
        rank_s[pl.ds(i * TB, TB)] = rank.astype(jnp.int32)

        counts_ref[...] += jnp.sum(mask, axis=0, keepdims=True)
        psum_ref[...] += jnp.sum(p, axis=0, keepdims=True)
        carry_ref[...] += jnp.sum(mask, axis=0, keepdims=True)

        pmax_ref[...] = jnp.broadcast_to(pmax, (TB, PW))

    @pl.when(i == _NSTEP - 1)
    def _():
        counts = counts_ref[...]
        nblk = jnp.floor((counts + (BT - 1)) * (1.0 / BT))
        ilt = (lax.broadcasted_iota(jnp.int32, (E, E), 0)
               < lax.broadcasted_iota(jnp.int32, (E, E), 1)).astype(jnp.float32)
        ile = (lax.broadcasted_iota(jnp.int32, (E, E), 0)
               <= lax.broadcasted_iota(jnp.int32, (E, E), 1)).astype(jnp.float32)
        off = jnp.dot(nblk * BT, ilt, preferred_element_type=jnp.float32)
        bend = jnp.dot(nblk, ile, preferred_element_type=jnp.float32)
        total = bend[0, E - 1]

        routes = routes_s[...]
        maskf = (lax.broadcasted_iota(jnp.int32, (T, E), 1)
                 == routes[:, None]).astype(jnp.float32)
        base = jnp.sum(maskf * off, axis=1)
        pos_ref[...] = rank_s[...] + base.astype(jnp.int32)

        biota = lax.broadcasted_iota(jnp.int32, (NB, E), 0).astype(jnp.float32)
        bexp = jnp.sum((biota >= jnp.broadcast_to(bend, (NB, E))
                        ).astype(jnp.float32), axis=1)
        bexp_ref[...] = jnp.minimum(bexp, E - 1).astype(jnp.int32)
        bval_ref[...] = (biota[:, 0] < total).astype(jnp.int32)


def _run_router(xf, W_r):
    nc = T // TB - 1
    return pl.pallas_call(
        _router_body,
        grid=(_NSTEP,),
        in_specs=[
            pl.BlockSpec((TB, H), lambda i: (jnp.minimum(i, nc), 0)),
            pl.BlockSpec((H, E), lambda i: (0, 0)),
        ],
        out_specs=[
            pl.BlockSpec((TB, E), lambda i: (jnp.minimum(i, nc), 0)),
            pl.BlockSpec((TB, PW), lambda i: (jnp.minimum(i, nc), 0)),
            pl.BlockSpec((1, E), lambda i: (0, 0)),
            pl.BlockSpec((1, E), lambda i: (0, 0)),
            pl.BlockSpec((T,), lambda i: (0,)),
            pl.BlockSpec((NB,), lambda i: (0,)),
            pl.BlockSpec((NB,), lambda i: (0,)),
        ],
        out_shape=[
            jax.ShapeDtypeStruct((T, E), jnp.float32),
            jax.ShapeDtypeStruct((T, PW), jnp.float32),
            jax.ShapeDtypeStruct((1, E), jnp.float32),
            jax.ShapeDtypeStruct((1, E), jnp.float32),
            jax.ShapeDtypeStruct((T,), jnp.int32),
            jax.ShapeDtypeStruct((NB,), jnp.int32),
            jax.ShapeDtypeStruct((NB,), jnp.int32),
        ],
        scratch_shapes=[
            pltpu.VMEM((1, E), jnp.float32),
            pltpu.VMEM((T,), jnp.int32),
            pltpu.VMEM((T,), jnp.int32),
        ],
    )(xf, W_r)


@functools.cache
def _sc_dispatch():
    mesh = plsc.VectorSubcoreMesh(
        core_axis_name="c", subcore_axis_name="s",
        num_cores=NC, num_subcores=NS)

    @functools.partial(
        pl.kernel,
        mesh=mesh,
        out_type=[
            jax.ShapeDtypeStruct((NSLOT, H), jnp.float32),
            jax.ShapeDtypeStruct((NSLOT, PW), jnp.float32),
        ],
        scratch_types=[
            pltpu.VMEM((TPW,), jnp.int32),
            pltpu.VMEM((TPW, H), jnp.float32),
            pltpu.VMEM((TPW, PW), jnp.float32),
            pltpu.SemaphoreType.DMA,
            pltpu.SemaphoreType.DMA,
        ],
    )
    def dispatch(x_hbm, pmax_hbm, pos_hbm, xs_hbm, ps_hbm, idx_v, rows_v,
                 pv, sem, sem2):
        wid = lax.axis_index("s") * NC + lax.axis_index("c")
        base = wid * TPW
        pltpu.sync_copy(pos_hbm.at[pl.ds(base, TPW)], idx_v)
        pltpu.sync_copy(x_hbm.at[pl.ds(base, TPW)], rows_v)
        pltpu.sync_copy(pmax_hbm.at[pl.ds(base, TPW)], pv)
        row_copy = pltpu.async_copy(rows_v, xs_hbm.at[idx_v], sem)
        p_copy = pltpu.async_copy(pv, ps_hbm.at[idx_v], sem2)
        row_copy.wait()
        p_copy.wait()

    return dispatch


def _mlp_body(bexp_ref, bval_ref, xs_ref, ps_ref, w1_ref, b1_ref, w2_ref,
              b2_ref, out_ref):
    b = pl.program_id(0)

    @pl.when(bval_ref[b] == 1)
    def _():
        e = bexp_ref[b]
        xb = xs_ref[...]
        pmax = ps_ref[:, :1]
        h = jnp.dot(xb, w1_ref[0], preferred_element_type=jnp.float32)
        h = jax.nn.gelu(h + b1_ref[pl.ds(e, 1), :])
        o = jnp.dot(h, w2_ref[0], preferred_element_type=jnp.float32)
        out_ref[...] = (o + b2_ref[pl.ds(e, 1), :]) * pmax


def _run_mlp(bexp, bval, xs, ps, W1, b1, W2, b2):
    grid_spec = pltpu.PrefetchScalarGridSpec(
        num_scalar_prefetch=2,
        grid=(NB,),
        in_specs=[
            pl.BlockSpec((BT, H), lambda b, be, bv: (b, 0)),
            pl.BlockSpec((BT, PW), lambda b, be, bv: (b, 0)),
            pl.BlockSpec((1, H, F), lambda b, be, bv: (be[b], 0, 0)),
            pl.BlockSpec((E, F), lambda b, be, bv: (0, 0)),
            pl.BlockSpec((1, F, H), lambda b, be, bv: (be[b], 0, 0)),
            pl.BlockSpec((E, H), lambda b, be, bv: (0, 0)),
        ],
        out_specs=pl.BlockSpec((BT, H), lambda b, be, bv: (b, 0)),
    )
    return pl.pallas_call(
        _mlp_body,
        grid_spec=grid_spec,
        out_shape=jax.ShapeDtypeStruct((NSLOT, H), jnp.float32),
        compiler_params=pltpu.CompilerParams(
            vmem_limit_bytes=120 * 1024 * 1024),
    )(bexp, bval, xs, ps, W1, b1, W2, b2)


@functools.cache
def _sc_combine():
    mesh = plsc.VectorSubcoreMesh(
        core_axis_name="c", subcore_axis_name="s",
        num_cores=NC, num_subcores=NS)

    @functools.partial(
        pl.kernel,
        mesh=mesh,
        out_type=jax.ShapeDtypeStruct((T, H), jnp.float32),
        scratch_types=[
            pltpu.VMEM((TPW,), jnp.int32),
            pltpu.VMEM((TPW, H), jnp.float32),
            pltpu.SemaphoreType.DMA,
        ],
    )
    def combine(ys_hbm, pos_hbm, y_hbm, idx_v, rows_v, sem):
        wid = lax.axis_index("s") * NC + lax.axis_index("c")
        base = wid * TPW
        pltpu.sync_copy(pos_hbm.at[pl.ds(base, TPW)], idx_v)
        pltpu.async_copy(ys_hbm.at[idx_v], rows_v, sem).wait()
        pltpu.sync_copy(rows_v, y_hbm.at[pl.ds(base, TPW)])

    return combine


@jax.jit
def kernel(x, W_r, W1, b1, W2, b2):
    original_shape = x.shape
    xf = x.reshape(T, H)
    logits, pmax16, counts, psum, pos, bexp, bval = _run_router(xf, W_r)
    xs, ps = _sc_dispatch()(xf, pmax16, pos)
    ys = _run_mlp(bexp, bval, xs, ps, W1, b1, W2, b2)
    y = _sc_combine()(ys, pos)
    return (y.reshape(original_shape), counts[0], psum[0],
            logits.reshape(original_shape[:-1] + (E,)))


# speedup vs baseline: 1.3610x; 1.0686x over previous
import functools

import jax
import jax.numpy as jnp
from jax import lax
from jax.experimental import pallas as pl
from jax.experimental.pallas import tpu as pltpu
from jax.experimental.pallas import tpu_sc as plsc

H = 768
F = 3072
E = 8
T = 2048
TB = 512
BT = 256
NB = 16
NSLOT = NB * BT
PW = 128
NC, NS = 2, 16
NW = NC * NS
TPW = T // NW


_NSTEP = T // TB + 1


def _router_body(x_ref, wr_ref, logits_ref, pmax_ref, counts_ref, psum_ref,
                 pos_ref, bexp_ref, bval_ref, carry_ref, routes_s, rank_s):
    i = pl.program_id(0)

    @pl.when(i == 0)
    def _():
        counts_ref[...] = jnp.zeros_like(counts_ref)
        psum_ref[...] = jnp.zeros_like(psum_ref)
        carry_ref[...] = jnp.zeros_like(carry_ref)

    @pl.when(i < _NSTEP - 1)
    def _():
        xb = x_ref[...]
        logits = jnp.dot(xb, wr_ref[...], preferred_element_type=jnp.float32)
        logits_ref[...] = logits
        m = jnp.max(logits, axis=1, keepdims=True)
        ex = jnp.exp(logits - m)
        p = ex / jnp.sum(ex, axis=1, keepdims=True)
        pmax = jnp.max(p, axis=1, keepdims=True)
        iota_e = lax.broadcasted_iota(jnp.int32, (TB, E), 1)
        routes = jnp.min(jnp.where(p == pmax, iota_e, E), axis=1)
        mask = (iota_e == routes[:, None]).astype(jnp.float32)

        ltri = (lax.broadcasted_iota(jnp.int32, (TB, TB), 0)
                > lax.broadcasted_iota(jnp.int32, (TB, TB), 1)
                ).astype(jnp.float32)
        exc = jnp.dot(ltri, mask, preferred_element_type=jnp.float32)
        rank = jnp.sum(mask * (carry_ref[...] + exc), axis=1)
        routes_s[pl.ds(i * TB, TB)] = routes
        rank_s[pl.ds(i * TB, TB)] = rank.astype(jnp.int32)

        counts_ref[...] += jnp.sum(mask, axis=0, keepdims=True)
        psum_ref[...] += jnp.sum(p, axis=0, keepdims=True)
        carry_ref[...] += jnp.sum(mask, axis=0, keepdims=True)

        pmax_ref[...] = jnp.broadcast_to(pmax, (TB, PW))

    @pl.when(i == _NSTEP - 1)
    def _():
        counts = counts_ref[...]
        nblk = jnp.floor((counts + (BT - 1)) * (1.0 / BT))
        ilt = (lax.broadcasted_iota(jnp.int32, (E, E), 0)
               < lax.broadcasted_iota(jnp.int32, (E, E), 1)).astype(jnp.float32)
        ile = (lax.broadcasted_iota(jnp.int32, (E, E), 0)
               <= lax.broadcasted_iota(jnp.int32, (E, E), 1)).astype(jnp.float32)
        off = jnp.dot(nblk * BT, ilt, preferred_element_type=jnp.float32)
        bend = jnp.dot(nblk, ile, preferred_element_type=jnp.float32)
        total = bend[0, E - 1]

        routes = routes_s[...]
        maskf = (lax.broadcasted_iota(jnp.int32, (T, E), 1)
                 == routes[:, None]).astype(jnp.float32)
        base = jnp.sum(maskf * off, axis=1)
        pos_ref[...] = rank_s[...] + base.astype(jnp.int32)

        biota = lax.broadcasted_iota(jnp.int32, (NB, E), 0).astype(jnp.float32)
        bexp = jnp.sum((biota >= jnp.broadcast_to(bend, (NB, E))
                        ).astype(jnp.float32), axis=1)
        bexp_ref[...] = jnp.minimum(bexp, E - 1).astype(jnp.int32)
        bval_ref[...] = (biota[:, 0] < total).astype(jnp.int32)


def _run_router(xf, W_r):
    nc = T // TB - 1
    return pl.pallas_call(
        _router_body,
        grid=(_NSTEP,),
        in_specs=[
            pl.BlockSpec((TB, H), lambda i: (jnp.minimum(i, nc), 0)),
            pl.BlockSpec((H, E), lambda i: (0, 0)),
        ],
        out_specs=[
            pl.BlockSpec((TB, E), lambda i: (jnp.minimum(i, nc), 0)),
            pl.BlockSpec((TB, PW), lambda i: (jnp.minimum(i, nc), 0)),
            pl.BlockSpec((1, E), lambda i: (0, 0)),
            pl.BlockSpec((1, E), lambda i: (0, 0)),
            pl.BlockSpec((T,), lambda i: (0,)),
            pl.BlockSpec((NB,), lambda i: (0,)),
            pl.BlockSpec((NB,), lambda i: (0,)),
        ],
        out_shape=[
            jax.ShapeDtypeStruct((T, E), jnp.float32),
            jax.ShapeDtypeStruct((T, PW), jnp.float32),
            jax.ShapeDtypeStruct((1, E), jnp.float32),
            jax.ShapeDtypeStruct((1, E), jnp.float32),
            jax.ShapeDtypeStruct((T,), jnp.int32),
            jax.ShapeDtypeStruct((NB,), jnp.int32),
            jax.ShapeDtypeStruct((NB,), jnp.int32),
        ],
        scratch_shapes=[
            pltpu.VMEM((1, E), jnp.float32),
            pltpu.VMEM((T,), jnp.int32),
            pltpu.VMEM((T,), jnp.int32),
        ],
    )(xf, W_r)


@functools.cache
def _sc_dispatch():
    mesh = plsc.VectorSubcoreMesh(
        core_axis_name="c", subcore_axis_name="s",
        num_cores=NC, num_subcores=NS)

    @functools.partial(
        pl.kernel,
        mesh=mesh,
        out_type=[
            jax.ShapeDtypeStruct((NSLOT, H), jnp.float32),
            jax.ShapeDtypeStruct((NSLOT, PW), jnp.float32),
        ],
        scratch_types=[
            pltpu.VMEM((TPW,), jnp.int32),
            pltpu.VMEM((TPW, H), jnp.float32),
            pltpu.VMEM((TPW, PW), jnp.float32),
            pltpu.SemaphoreType.DMA,
            pltpu.SemaphoreType.DMA,
        ],
    )
    def dispatch(x_hbm, pmax_hbm, pos_hbm, xs_hbm, ps_hbm, idx_v, rows_v,
                 pv, sem, sem2):
        wid = lax.axis_index("s") * NC + lax.axis_index("c")
        base = wid * TPW
        pltpu.sync_copy(pos_hbm.at[pl.ds(base, TPW)], idx_v)
        pltpu.sync_copy(x_hbm.at[pl.ds(base, TPW)], rows_v)
        pltpu.sync_copy(pmax_hbm.at[pl.ds(base, TPW)], pv)
        row_copy = pltpu.async_copy(rows_v, xs_hbm.at[idx_v], sem)
        p_copy = pltpu.async_copy(pv, ps_hbm.at[idx_v], sem2)
        row_copy.wait()
        p_copy.wait()

    return dispatch


_NFC = 4
_FC = F // _NFC


def _mlp_body(bexp_ref, bval_ref, xs_ref, ps_ref, w1_ref, b1_ref, w2_ref,
              b2_ref, out_ref):
    b = pl.program_id(0)

    @pl.when(bval_ref[b] == 1)
    def _():
        e = bexp_ref[b]
        xb = xs_ref[...]
        pmax = ps_ref[:, :1]
        o = jnp.zeros((BT, H), jnp.float32)
        for k in range(_NFC):
            lo, hi = k * _FC, (k + 1) * _FC
            hk = jnp.dot(xb, w1_ref[0, :, lo:hi],
                         preferred_element_type=jnp.float32)
            gk = jax.nn.gelu(hk + b1_ref[pl.ds(e, 1), lo:hi])
            o += jnp.dot(gk, w2_ref[0, lo:hi, :],
                         preferred_element_type=jnp.float32)
        out_ref[...] = (o + b2_ref[pl.ds(e, 1), :]) * pmax


def _run_mlp(bexp, bval, xs, ps, W1, b1, W2, b2):
    grid_spec = pltpu.PrefetchScalarGridSpec(
        num_scalar_prefetch=2,
        grid=(NB,),
        in_specs=[
            pl.BlockSpec((BT, H), lambda b, be, bv: (b, 0)),
            pl.BlockSpec((BT, PW), lambda b, be, bv: (b, 0)),
            pl.BlockSpec((1, H, F), lambda b, be, bv: (be[b], 0, 0)),
            pl.BlockSpec((E, F), lambda b, be, bv: (0, 0)),
            pl.BlockSpec((1, F, H), lambda b, be, bv: (be[b], 0, 0)),
            pl.BlockSpec((E, H), lambda b, be, bv: (0, 0)),
        ],
        out_specs=pl.BlockSpec((BT, H), lambda b, be, bv: (b, 0)),
    )
    return pl.pallas_call(
        _mlp_body,
        grid_spec=grid_spec,
        out_shape=jax.ShapeDtypeStruct((NSLOT, H), jnp.float32),
        compiler_params=pltpu.CompilerParams(
            vmem_limit_bytes=120 * 1024 * 1024),
    )(bexp, bval, xs, ps, W1, b1, W2, b2)


@functools.cache
def _sc_combine():
    mesh = plsc.VectorSubcoreMesh(
        core_axis_name="c", subcore_axis_name="s",
        num_cores=NC, num_subcores=NS)

    @functools.partial(
        pl.kernel,
        mesh=mesh,
        out_type=jax.ShapeDtypeStruct((T, H), jnp.float32),
        scratch_types=[
            pltpu.VMEM((TPW,), jnp.int32),
            pltpu.VMEM((TPW, H), jnp.float32),
            pltpu.SemaphoreType.DMA,
        ],
    )
    def combine(ys_hbm, pos_hbm, y_hbm, idx_v, rows_v, sem):
        wid = lax.axis_index("s") * NC + lax.axis_index("c")
        base = wid * TPW
        pltpu.sync_copy(pos_hbm.at[pl.ds(base, TPW)], idx_v)
        pltpu.async_copy(ys_hbm.at[idx_v], rows_v, sem).wait()
        pltpu.sync_copy(rows_v, y_hbm.at[pl.ds(base, TPW)])

    return combine


@jax.jit
def kernel(x, W_r, W1, b1, W2, b2):
    original_shape = x.shape
    xf = x.reshape(T, H)
    logits, pmax16, counts, psum, pos, bexp, bval = _run_router(xf, W_r)
    xs, ps = _sc_dispatch()(xf, pmax16, pos)
    ys = _run_mlp(bexp, bval, xs, ps, W1, b1, W2, b2)
    y = _sc_combine()(ys, pos)
    return (y.reshape(original_shape), counts[0], psum[0],
            logits.reshape(original_shape[:-1] + (E,)))


# speedup vs baseline: 1.4719x; 1.0815x over previous
import functools

import jax
import jax.numpy as jnp
from jax import lax
from jax.experimental import pallas as pl
from jax.experimental.pallas import tpu as pltpu
from jax.experimental.pallas import tpu_sc as plsc

H = 768
F = 3072
E = 8
T = 2048
TB = 512
BT = 512
NB = 12
NSLOT = NB * BT
PW = 128
NC, NS = 2, 16
NW = NC * NS
TPW = T // NW


_NSTEP = T // TB + 1


def _router_body(x_ref, wr_ref, logits_ref, pmax_ref, counts_ref, psum_ref,
                 pos_ref, bexp_ref, bval_ref, carry_ref, routes_s, rank_s):
    i = pl.program_id(0)

    @pl.when(i == 0)
    def _():
        counts_ref[...] = jnp.zeros_like(counts_ref)
        psum_ref[...] = jnp.zeros_like(psum_ref)
        carry_ref[...] = jnp.zeros_like(carry_ref)

    @pl.when(i < _NSTEP - 1)
    def _():
        xb = x_ref[...]
        logits = jnp.dot(xb, wr_ref[...], preferred_element_type=jnp.float32)
        logits_ref[...] = logits
        m = jnp.max(logits, axis=1, keepdims=True)
        ex = jnp.exp(logits - m)
        p = ex / jnp.sum(ex, axis=1, keepdims=True)
        pmax = jnp.max(p, axis=1, keepdims=True)
        iota_e = lax.broadcasted_iota(jnp.int32, (TB, E), 1)
        routes = jnp.min(jnp.where(p == pmax, iota_e, E), axis=1)
        mask = (iota_e == routes[:, None]).astype(jnp.float32)

        ltri = (lax.broadcasted_iota(jnp.int32, (TB, TB), 0)
                > lax.broadcasted_iota(jnp.int32, (TB, TB), 1)
                ).astype(jnp.float32)
        exc = jnp.dot(ltri, mask, preferred_element_type=jnp.float32)
        rank = jnp.sum(mask * (carry_ref[...] + exc), axis=1)
        routes_s[pl.ds(i * TB, TB)] = routes
        rank_s[pl.ds(i * TB, TB)] = rank.astype(jnp.int32)

        counts_ref[...] += jnp.sum(mask, axis=0, keepdims=True)
        psum_ref[...] += jnp.sum(p, axis=0, keepdims=True)
        carry_ref[...] += jnp.sum(mask, axis=0, keepdims=True)

        pmax_ref[...] = jnp.broadcast_to(pmax, (TB, PW))

    @pl.when(i == _NSTEP - 1)
    def _():
        counts = counts_ref[...]
        nblk = jnp.floor((counts + (BT - 1)) * (1.0 / BT))
        ilt = (lax.broadcasted_iota(jnp.int32, (E, E), 0)
               < lax.broadcasted_iota(jnp.int32, (E, E), 1)).astype(jnp.float32)
        ile = (lax.broadcasted_iota(jnp.int32, (E, E), 0)
               <= lax.broadcasted_iota(jnp.int32, (E, E), 1)).astype(jnp.float32)
        off = jnp.dot(nblk * BT, ilt, preferred_element_type=jnp.float32)
        bend = jnp.dot(nblk, ile, preferred_element_type=jnp.float32)
        total = bend[0, E - 1]

        routes = routes_s[...]
        maskf = (lax.broadcasted_iota(jnp.int32, (T, E), 1)
                 == routes[:, None]).astype(jnp.float32)
        base = jnp.sum(maskf * off, axis=1)
        pos_ref[...] = rank_s[...] + base.astype(jnp.int32)

        biota = lax.broadcasted_iota(jnp.int32, (NB, E), 0).astype(jnp.float32)
        bexp = jnp.sum((biota >= jnp.broadcast_to(bend, (NB, E))
                        ).astype(jnp.float32), axis=1)
        bexp_ref[...] = jnp.minimum(bexp, E - 1).astype(jnp.int32)
        bval_ref[...] = (biota[:, 0] < total).astype(jnp.int32)


def _run_router(xf, W_r):
    nc = T // TB - 1
    return pl.pallas_call(
        _router_body,
        grid=(_NSTEP,),
        in_specs=[
            pl.BlockSpec((TB, H), lambda i: (jnp.minimum(i, nc), 0)),
            pl.BlockSpec((H, E), lambda i: (0, 0)),
        ],
        out_specs=[
            pl.BlockSpec((TB, E), lambda i: (jnp.minimum(i, nc), 0)),
            pl.BlockSpec((TB, PW), lambda i: (jnp.minimum(i, nc), 0)),
            pl.BlockSpec((1, E), lambda i: (0, 0)),
            pl.BlockSpec((1, E), lambda i: (0, 0)),
            pl.BlockSpec((T,), lambda i: (0,)),
            pl.BlockSpec((NB,), lambda i: (0,)),
            pl.BlockSpec((NB,), lambda i: (0,)),
        ],
        out_shape=[
            jax.ShapeDtypeStruct((T, E), jnp.float32),
            jax.ShapeDtypeStruct((T, PW), jnp.float32),
            jax.ShapeDtypeStruct((1, E), jnp.float32),
            jax.ShapeDtypeStruct((1, E), jnp.float32),
            jax.ShapeDtypeStruct((T,), jnp.int32),
            jax.ShapeDtypeStruct((NB,), jnp.int32),
            jax.ShapeDtypeStruct((NB,), jnp.int32),
        ],
        scratch_shapes=[
            pltpu.VMEM((1, E), jnp.float32),
            pltpu.VMEM((T,), jnp.int32),
            pltpu.VMEM((T,), jnp.int32),
        ],
    )(xf, W_r)


@functools.cache
def _sc_dispatch():
    mesh = plsc.VectorSubcoreMesh(
        core_axis_name="c", subcore_axis_name="s",
        num_cores=NC, num_subcores=NS)

    @functools.partial(
        pl.kernel,
        mesh=mesh,
        out_type=[
            jax.ShapeDtypeStruct((NSLOT, H), jnp.float32),
            jax.ShapeDtypeStruct((NSLOT, PW), jnp.float32),
        ],
        scratch_types=[
            pltpu.VMEM((TPW,), jnp.int32),
            pltpu.VMEM((TPW, H), jnp.float32),
            pltpu.VMEM((TPW, PW), jnp.float32),
            pltpu.SemaphoreType.DMA,
            pltpu.SemaphoreType.DMA,
        ],
    )
    def dispatch(x_hbm, pmax_hbm, pos_hbm, xs_hbm, ps_hbm, idx_v, rows_v,
                 pv, sem, sem2):
        wid = lax.axis_index("s") * NC + lax.axis_index("c")
        base = wid * TPW
        pltpu.sync_copy(pos_hbm.at[pl.ds(base, TPW)], idx_v)
        pltpu.sync_copy(x_hbm.at[pl.ds(base, TPW)], rows_v)
        pltpu.sync_copy(pmax_hbm.at[pl.ds(base, TPW)], pv)
        row_copy = pltpu.async_copy(rows_v, xs_hbm.at[idx_v], sem)
        p_copy = pltpu.async_copy(pv, ps_hbm.at[idx_v], sem2)
        row_copy.wait()
        p_copy.wait()

    return dispatch


_NFC = 4
_FC = F // _NFC


def _mlp_body(bexp_ref, bval_ref, xs_ref, ps_ref, w1_ref, b1_ref, w2_ref,
              b2_ref, out_ref):
    b = pl.program_id(0)

    @pl.when(bval_ref[b] == 1)
    def _():
        e = bexp_ref[b]
        xb = xs_ref[...]
        pmax = ps_ref[:, :1]
        o = jnp.zeros((BT, H), jnp.float32)
        for k in range(_NFC):
            lo, hi = k * _FC, (k + 1) * _FC
            hk = jnp.dot(xb, w1_ref[0, :, lo:hi],
                         preferred_element_type=jnp.float32)
            gk = jax.nn.gelu(hk + b1_ref[pl.ds(e, 1), lo:hi])
            o += jnp.dot(gk, w2_ref[0, lo:hi, :],
                         preferred_element_type=jnp.float32)
        out_ref[...] = (o + b2_ref[pl.ds(e, 1), :]) * pmax


def _run_mlp(bexp, bval, xs, ps, W1, b1, W2, b2):
    grid_spec = pltpu.PrefetchScalarGridSpec(
        num_scalar_prefetch=2,
        grid=(NB,),
        in_specs=[
            pl.BlockSpec((BT, H), lambda b, be, bv: (b, 0)),
            pl.BlockSpec((BT, PW), lambda b, be, bv: (b, 0)),
            pl.BlockSpec((1, H, F), lambda b, be, bv: (be[b], 0, 0)),
            pl.BlockSpec((E, F), lambda b, be, bv: (0, 0)),
            pl.BlockSpec((1, F, H), lambda b, be, bv: (be[b], 0, 0)),
            pl.BlockSpec((E, H), lambda b, be, bv: (0, 0)),
        ],
        out_specs=pl.BlockSpec((BT, H), lambda b, be, bv: (b, 0)),
    )
    return pl.pallas_call(
        _mlp_body,
        grid_spec=grid_spec,
        out_shape=jax.ShapeDtypeStruct((NSLOT, H), jnp.float32),
        compiler_params=pltpu.CompilerParams(
            vmem_limit_bytes=120 * 1024 * 1024),
    )(bexp, bval, xs, ps, W1, b1, W2, b2)


@functools.cache
def _sc_combine():
    mesh = plsc.VectorSubcoreMesh(
        core_axis_name="c", subcore_axis_name="s",
        num_cores=NC, num_subcores=NS)

    @functools.partial(
        pl.kernel,
        mesh=mesh,
        out_type=jax.ShapeDtypeStruct((T, H), jnp.float32),
        scratch_types=[
            pltpu.VMEM((TPW,), jnp.int32),
            pltpu.VMEM((TPW, H), jnp.float32),
            pltpu.SemaphoreType.DMA,
        ],
    )
    def combine(ys_hbm, pos_hbm, y_hbm, idx_v, rows_v, sem):
        wid = lax.axis_index("s") * NC + lax.axis_index("c")
        base = wid * TPW
        pltpu.sync_copy(pos_hbm.at[pl.ds(base, TPW)], idx_v)
        pltpu.async_copy(ys_hbm.at[idx_v], rows_v, sem).wait()
        pltpu.sync_copy(rows_v, y_hbm.at[pl.ds(base, TPW)])

    return combine


@jax.jit
def kernel(x, W_r, W1, b1, W2, b2):
    original_shape = x.shape
    xf = x.reshape(T, H)
    logits, pmax16, counts, psum, pos, bexp, bval = _run_router(xf, W_r)
    xs, ps = _sc_dispatch()(xf, pmax16, pos)
    ys = _run_mlp(bexp, bval, xs, ps, W1, b1, W2, b2)
    y = _sc_combine()(ys, pos)
    return (y.reshape(original_shape), counts[0], psum[0],
            logits.reshape(original_shape[:-1] + (E,)))
